# Initial kernel scaffold; baseline (speedup 1.0000x reference)
import jax
import jax.numpy as jnp
from jax.experimental import pallas as pl


def kernel(x, edge_index, edge_attr, Wq, bq, Wk, bk, Wv, bv, Wek, bek, Wev, bev, W1, b1, W2, b2, ln_src_w, ln_src_b, ln_edge_w, ln_edge_b, ln_ffn_w, ln_ffn_b):
    raise NotImplementedError("write your pallas kernel here")


# speedup vs baseline: 17.9452x; 17.9452x over previous
import functools

import jax
import jax.numpy as jnp
from jax import lax
from jax.experimental import pallas as pl
from jax.experimental.pallas import tpu as pltpu
from jax.experimental.pallas import tpu_sc as plsc

HID = 128
HEADS = 8
HEAD_DIM = HID // HEADS
N_NODES = 10000
N_EDGES = 320000

NC = 2
NS = 16
NW = NC * NS
EPW = N_EDGES // NW
CH = 80
NCHUNK = EPW // CH
SCH = 80
NSCHUNK = EPW // SCH
NPAD = 10240
ROWS_PER_TILE = NPAD // NS

_F32 = jnp.float32
_HIGHEST = jax.lax.Precision.HIGHEST


def _mm_t(a, w):
    return jax.lax.dot_general(
        a, w, (((1,), (1,)), ((), ())),
        precision=_HIGHEST, preferred_element_type=_F32)


def _mm(a, b):
    return jax.lax.dot_general(
        a, b, (((1,), (0,)), ((), ())),
        precision=_HIGHEST, preferred_element_type=_F32)


def _ln(x, w, b):
    mu = jnp.mean(x, axis=-1, keepdims=True)
    var = jnp.mean((x - mu) ** 2, axis=-1, keepdims=True)
    return (x - mu) / jnp.sqrt(var + 1e-5) * w + b


def _node_prep_body(x_ref, wq_ref, bq_ref, wk_ref, bk_ref, wv_ref, bv_ref,
                    lw_ref, lb_ref, xn_ref, q_ref, k_ref, v_ref):
    xn = _ln(x_ref[...], lw_ref[...], lb_ref[...])
    xn_ref[...] = xn
    q_ref[...] = (_mm_t(xn, wq_ref[...]) + bq_ref[...]) * (1.0 / jnp.sqrt(jnp.float32(HEAD_DIM)))
    k_ref[...] = _mm_t(xn, wk_ref[...]) + bk_ref[...]
    v_ref[...] = _mm_t(xn, wv_ref[...]) + bv_ref[...]


def _node_prep(x, Wq, bq, Wk, bk, Wv, bv, lw, lb):
    out = jax.ShapeDtypeStruct((N_NODES, HID), _F32)
    return pl.pallas_call(
        _node_prep_body,
        out_shape=(out, out, out, out),
    )(x, Wq, bq.reshape(1, HID), Wk, bk.reshape(1, HID),
      Wv, bv.reshape(1, HID), lw.reshape(1, HID), lb.reshape(1, HID))


def _sc_gather(Qs, K, V, src, dst):
    mesh = plsc.VectorSubcoreMesh(core_axis_name="c", subcore_axis_name="s")
    erows = jax.ShapeDtypeStruct((N_EDGES, HID), _F32)

    @functools.partial(
        pl.kernel,
        out_type=(erows, erows, erows),
        mesh=mesh,
        scratch_types=[
            pltpu.VMEM((CH,), jnp.int32),
            pltpu.VMEM((CH,), jnp.int32),
            pltpu.VMEM((CH, HID), _F32),
            pltpu.SemaphoreType.DMA,
        ],
    )
    def gather_kernel(q_hbm, k_hbm, v_hbm, src_hbm, dst_hbm,
                      qd_hbm, kn_hbm, vn_hbm, si_v, di_v, rows_v, sem):
        wid = lax.axis_index("s") * NC + lax.axis_index("c")

        def body(i, carry):
            base = wid * EPW + i * CH
            pltpu.sync_copy(src_hbm.at[pl.ds(base, CH)], si_v)
            pltpu.sync_copy(dst_hbm.at[pl.ds(base, CH)], di_v)
            pltpu.async_copy(q_hbm.at[di_v], rows_v, sem).wait()
            pltpu.sync_copy(rows_v, qd_hbm.at[pl.ds(base, CH)])
            pltpu.async_copy(k_hbm.at[si_v], rows_v, sem).wait()
            pltpu.sync_copy(rows_v, kn_hbm.at[pl.ds(base, CH)])
            pltpu.async_copy(v_hbm.at[si_v], rows_v, sem).wait()
            pltpu.sync_copy(rows_v, vn_hbm.at[pl.ds(base, CH)])
            return carry

        lax.fori_loop(0, NCHUNK, body, 0)

    return gather_kernel(Qs, K, V, src, dst)


def _edge_body(ea_ref, qd_ref, kn_ref, vn_ref, dst_ref, wek_ref, bek_ref,
               wev_ref, bev_ref, lw_ref, lb_ref, sel_ref, bc_ref, pk_ref,
               msg_ref, ex_ref):
    en = _ln(ea_ref[...], lw_ref[...], lb_ref[...])
    ek = _mm_t(en, wek_ref[...]) + bek_ref[...]
    prod = qd_ref[...] * (kn_ref[...] + ek)
    ex16 = jnp.exp(_mm(prod, sel_ref[...]))
    ev = _mm_t(en, wev_ref[...]) + bev_ref[...]
    exb = _mm(ex16, bc_ref[...])
    msg_ref[...] = (vn_ref[...] + ev) * exb
    be = ea_ref.shape[0]
    slot = (dst_ref[0, 0, :] % 8).reshape(be, 1)
    col = lax.broadcasted_iota(jnp.int32, (be, HID), 1)
    exhc = _mm(ex16, pk_ref[...])
    ex_ref[...] = jnp.where((col // HEAD_DIM) == slot, exhc, 0.0)


def _edge_math(edge_attr, QD, KN, VN, dst, Wek, bek, Wev, bev, lw, lb):
    BE = 2000
    grid = N_EDGES // BE
    eye = jnp.concatenate([jnp.eye(HEADS, dtype=_F32),
                           jnp.zeros((HEADS, HEADS), _F32)], axis=1)
    sel = jnp.repeat(eye, HEAD_DIM, axis=0)
    bc = jnp.concatenate([jnp.repeat(jnp.eye(HEADS, dtype=_F32), HEAD_DIM, axis=1),
                          jnp.zeros((HEADS, HID), _F32)], axis=0)
    colv = jnp.arange(HID) % (2 * HEADS)
    pk = (colv[None, :] == jnp.arange(2 * HEADS)[:, None]).astype(_F32)
    pk = pk.at[HEADS:].set(0.0)
    row_spec = pl.BlockSpec((BE, HID), lambda i: (i, 0))
    w_spec = pl.BlockSpec((HID, HID), lambda i: (0, 0))
    b_spec = pl.BlockSpec((1, HID), lambda i: (0, 0))
    return pl.pallas_call(
        _edge_body,
        grid=(grid,),
        in_specs=[row_spec, row_spec, row_spec, row_spec,
                  pl.BlockSpec((1, 1, BE), lambda i: (i, 0, 0)),
                  w_spec, b_spec, w_spec, b_spec, b_spec, b_spec,
                  pl.BlockSpec((HID, 2 * HEADS), lambda i: (0, 0)),
                  pl.BlockSpec((2 * HEADS, HID), lambda i: (0, 0)),
                  pl.BlockSpec((2 * HEADS, HID), lambda i: (0, 0))],
        out_specs=(row_spec, row_spec),
        out_shape=(jax.ShapeDtypeStruct((N_EDGES, HID), _F32),
                   jax.ShapeDtypeStruct((N_EDGES, HID), _F32)),
    )(edge_attr, QD, KN, VN, dst.reshape(grid, 1, BE), Wek, bek.reshape(1, HID),
      Wev, bev.reshape(1, HID), lw.reshape(1, HID), lb.reshape(1, HID),
      sel, bc, pk)


DPAD = NPAD // 8
DROWS_PER_TILE = DPAD // NS


def _sc_scatter(MSG, EX, dst):
    mesh = plsc.VectorSubcoreMesh(core_axis_name="c", subcore_axis_name="s")

    @functools.partial(
        pl.kernel,
        out_type=(jax.ShapeDtypeStruct((NC, NPAD, HID), _F32),
                  jax.ShapeDtypeStruct((NC, DPAD, HID), _F32)),
        mesh=mesh,
        scratch_types=[
            pltpu.VMEM((SCH,), jnp.int32),
            pltpu.VMEM((SCH,), jnp.int32),
            pltpu.VMEM((SCH,), jnp.int32),
            pltpu.VMEM((SCH, HID), _F32),
            pltpu.VMEM((SCH, HID), _F32),
            pltpu.VMEM_SHARED((NPAD, HID), _F32),
            pltpu.VMEM_SHARED((DPAD, HID), _F32),
            pltpu.SemaphoreType.DMA,
        ],
    )
    def scatter_kernel(msg_hbm, ex_hbm, dst_hbm, on_hbm, od_hbm,
                       di_v, dp_v, zi_v, m_v, e_v, accn_s, accd_s, sem):
        cid = lax.axis_index("c")
        sid = lax.axis_index("s")
        wid = sid * NC + cid

        zeros16 = jnp.zeros((16,), _F32)

        def zrow(i, carry):
            for j in range(HID // 16):
                m_v[i, pl.ds(j * 16, 16)] = zeros16
                e_v[i, pl.ds(j * 16, 16)] = zeros16
            return carry

        off = sid * ROWS_PER_TILE
        doff = sid * DROWS_PER_TILE
        lax.fori_loop(0, SCH, zrow, 0)

        iota16 = lax.iota(jnp.int32, 16)

        def _fill_zidx(base):
            for t in range(SCH // 16):
                zi_v[pl.ds(t * 16, 16)] = iota16 + (base + t * 16)

        for j in range(ROWS_PER_TILE // SCH):
            _fill_zidx(off + j * SCH)
            pltpu.sync_copy(m_v, accn_s.at[zi_v])
        _fill_zidx(doff)
        pltpu.sync_copy(e_v, accd_s.at[zi_v])
        plsc.subcore_barrier()

        def body(i, carry):
            base = wid * EPW + i * SCH
            pltpu.sync_copy(dst_hbm.at[pl.ds(base, SCH)], di_v)
            pltpu.sync_copy(msg_hbm.at[pl.ds(base, SCH)], m_v)
            pltpu.sync_copy(ex_hbm.at[pl.ds(base, SCH)], e_v)
            for t in range(SCH // 16):
                dp_v[pl.ds(t * 16, 16)] = lax.shift_right_logical(
                    di_v[pl.ds(t * 16, 16)], 3)
            pltpu.sync_copy(m_v, accn_s.at[di_v], add=True)
            pltpu.sync_copy(e_v, accd_s.at[dp_v], add=True)
            return carry

        lax.fori_loop(0, NSCHUNK, body, 0)
        plsc.subcore_barrier()

        for j in range(ROWS_PER_TILE // SCH):
            _fill_zidx(off + j * SCH)
            pltpu.async_copy(accn_s.at[zi_v], m_v, sem).wait()
            pltpu.sync_copy(m_v, on_hbm.at[cid, pl.ds(off + j * SCH, SCH)])
        _fill_zidx(doff)
        pltpu.async_copy(accd_s.at[zi_v], e_v, sem).wait()
        pltpu.sync_copy(e_v, od_hbm.at[cid, pl.ds(doff, SCH)])

    return scatter_kernel(MSG, EX, dst)


def _final_body(xn_ref, pn_ref, pd_ref, w1_ref, b1_ref, w2_ref, b2_ref,
                lw_ref, lb_ref, bc_ref, out_ref):
    num = pn_ref[0] + pn_ref[1]
    den = pd_ref[0] + pd_ref[1]
    denb = _mm(den, bc_ref[...])
    x_dst = xn_ref[...] + num / (denb + 1e-16)
    h = _ln(x_dst, lw_ref[...], lb_ref[...])
    t = jnp.maximum(_mm_t(h, w1_ref[...]) + b1_ref[...], 0.0)
    out_ref[...] = x_dst + _mm_t(t, w2_ref[...]) + b2_ref[...]


def _finalize(x_n, PN, PD, W1, b1, W2, b2, lw, lb):
    BN = 2000
    grid = N_NODES // BN
    DW = 2 * HEADS
    bc = jnp.concatenate([jnp.repeat(jnp.eye(HEADS, dtype=_F32), HEAD_DIM, axis=1),
                          jnp.zeros((HEADS, HID), _F32)], axis=0)
    return pl.pallas_call(
        _final_body,
        grid=(grid,),
        in_specs=[pl.BlockSpec((BN, HID), lambda i: (i, 0)),
                  pl.BlockSpec((NC, BN, HID), lambda i: (0, i, 0)),
                  pl.BlockSpec((NC, BN, DW), lambda i: (0, i, 0)),
                  pl.BlockSpec((4 * HID, HID), lambda i: (0, 0)),
                  pl.BlockSpec((1, 4 * HID), lambda i: (0, 0)),
                  pl.BlockSpec((HID, 4 * HID), lambda i: (0, 0)),
                  pl.BlockSpec((1, HID), lambda i: (0, 0)),
                  pl.BlockSpec((1, HID), lambda i: (0, 0)),
                  pl.BlockSpec((1, HID), lambda i: (0, 0)),
                  pl.BlockSpec((DW, HID), lambda i: (0, 0))],
        out_specs=pl.BlockSpec((BN, HID), lambda i: (i, 0)),
        out_shape=jax.ShapeDtypeStruct((N_NODES, HID), _F32),
    )(x_n, PN, PD, W1, b1.reshape(1, 4 * HID), W2, b2.reshape(1, HID),
      lw.reshape(1, HID), lb.reshape(1, HID), bc)


def kernel(x, edge_index, edge_attr, Wq, bq, Wk, bk, Wv, bv, Wek, bek, Wev, bev,
           W1, b1, W2, b2, ln_src_w, ln_src_b, ln_edge_w, ln_edge_b,
           ln_ffn_w, ln_ffn_b):
    src = edge_index[0].astype(jnp.int32)
    dst = edge_index[1].astype(jnp.int32)
    x_n, Qs, K, V = _node_prep(x, Wq, bq, Wk, bk, Wv, bv, ln_src_w, ln_src_b)
    QD, KN, VN = _sc_gather(Qs, K, V, src, dst)
    MSG, EX = _edge_math(edge_attr, QD, KN, VN, dst, Wek, bek, Wev, bev,
                         ln_edge_w, ln_edge_b)
    PN, PD2 = _sc_scatter(MSG, EX, dst)
    PD = PD2.reshape(NC, NPAD, 2 * HEADS)
    return _finalize(x_n, PN, PD, W1, b1, W2, b2, ln_ffn_w, ln_ffn_b)


# speedup vs baseline: 37.2001x; 2.0730x over previous
import functools

import jax
import jax.numpy as jnp
from jax import lax
from jax.experimental import pallas as pl
from jax.experimental.pallas import tpu as pltpu
from jax.experimental.pallas import tpu_sc as plsc

HID = 128
HEADS = 8
HEAD_DIM = HID // HEADS
N_NODES = 10000
N_EDGES = 320000

NC = 2
NS = 16
NW = NC * NS
EPW = N_EDGES // NW
CH = 80
NCHUNK = EPW // CH
SCH = 80
NSCHUNK = EPW // SCH
NPAD = 10240
ROWS_PER_TILE = NPAD // NS

_F32 = jnp.float32
_HIGH = jax.lax.Precision.DEFAULT


def _mm_t(a, w):
    return jax.lax.dot_general(
        a, w, (((1,), (1,)), ((), ())),
        precision=_HIGH, preferred_element_type=_F32)


def _mm(a, b):
    return jax.lax.dot_general(
        a, b, (((1,), (0,)), ((), ())),
        precision=_HIGH, preferred_element_type=_F32)


def _ln(x, w, b):
    mu = jnp.mean(x, axis=-1, keepdims=True)
    var = jnp.mean((x - mu) ** 2, axis=-1, keepdims=True)
    return (x - mu) / jnp.sqrt(var + 1e-5) * w + b


def _node_prep_body(x_ref, wq_ref, bq_ref, wk_ref, bk_ref, wv_ref, bv_ref,
                    lw_ref, lb_ref, xn_ref, q_ref, k_ref, v_ref):
    xn = _ln(x_ref[...], lw_ref[...], lb_ref[...])
    xn_ref[...] = xn
    q_ref[...] = (_mm_t(xn, wq_ref[...]) + bq_ref[...]) * (1.0 / jnp.sqrt(jnp.float32(HEAD_DIM)))
    k_ref[...] = _mm_t(xn, wk_ref[...]) + bk_ref[...]
    v_ref[...] = _mm_t(xn, wv_ref[...]) + bv_ref[...]


def _node_prep(x, Wq, bq, Wk, bk, Wv, bv, lw, lb):
    out = jax.ShapeDtypeStruct((N_NODES, HID), _F32)
    return pl.pallas_call(
        _node_prep_body,
        out_shape=(out, out, out, out),
    )(x, Wq, bq.reshape(1, HID), Wk, bk.reshape(1, HID),
      Wv, bv.reshape(1, HID), lw.reshape(1, HID), lb.reshape(1, HID))


def _sc_gather(Qs, K, V, src, dst):
    mesh = plsc.VectorSubcoreMesh(core_axis_name="c", subcore_axis_name="s")
    erows = jax.ShapeDtypeStruct((N_EDGES, HID), _F32)

    @functools.partial(
        pl.kernel,
        out_type=(erows, erows, erows),
        mesh=mesh,
        scratch_types=[
            pltpu.VMEM((EPW,), jnp.int32),
            pltpu.VMEM((EPW,), jnp.int32),
            pltpu.VMEM((CH,), jnp.int32),
            pltpu.VMEM((CH,), jnp.int32),
            pltpu.VMEM((CH, HID), _F32),
            pltpu.VMEM((CH, HID), _F32),
            pltpu.VMEM((CH, HID), _F32),
            pltpu.SemaphoreType.DMA,
            pltpu.SemaphoreType.DMA,
            pltpu.SemaphoreType.DMA,
        ],
    )
    def gather_kernel(q_hbm, k_hbm, v_hbm, src_hbm, dst_hbm,
                      qd_hbm, kn_hbm, vn_hbm, si_v, di_v, sc_v, dc_v,
                      bq_v, bk_v, bv_v, sq, sk, sv):
        wid = lax.axis_index("s") * NC + lax.axis_index("c")
        base0 = wid * EPW
        pltpu.sync_copy(src_hbm.at[pl.ds(base0, EPW)], si_v)
        pltpu.sync_copy(dst_hbm.at[pl.ds(base0, EPW)], di_v)

        def body(i, carry):
            base = base0 + i * CH
            for t in range(CH // 16):
                sc_v[pl.ds(t * 16, 16)] = si_v[pl.ds(i * CH + t * 16, 16)]
                dc_v[pl.ds(t * 16, 16)] = di_v[pl.ds(i * CH + t * 16, 16)]
            cq = pltpu.async_copy(q_hbm.at[dc_v], bq_v, sq)
            ck = pltpu.async_copy(k_hbm.at[sc_v], bk_v, sk)
            cv = pltpu.async_copy(v_hbm.at[sc_v], bv_v, sv)
            cq.wait()
            pltpu.sync_copy(bq_v, qd_hbm.at[pl.ds(base, CH)])
            ck.wait()
            pltpu.sync_copy(bk_v, kn_hbm.at[pl.ds(base, CH)])
            cv.wait()
            pltpu.sync_copy(bv_v, vn_hbm.at[pl.ds(base, CH)])
            return carry

        lax.fori_loop(0, NCHUNK, body, 0)

    return gather_kernel(Qs, K, V, src, dst)


def _edge_body(ea_ref, qd_ref, kn_ref, vn_ref, dst_ref, wek_ref, bek_ref,
               wev_ref, bev_ref, lw_ref, lb_ref, sel_ref, bc_ref, pk_ref,
               msg_ref, ex_ref):
    en = _ln(ea_ref[...], lw_ref[...], lb_ref[...])
    ek = _mm_t(en, wek_ref[...]) + bek_ref[...]
    prod = qd_ref[...] * (kn_ref[...] + ek)
    ex16 = jnp.exp(_mm(prod, sel_ref[...]))
    ev = _mm_t(en, wev_ref[...]) + bev_ref[...]
    exb = _mm(ex16, bc_ref[...])
    msg_ref[...] = (vn_ref[...] + ev) * exb
    be = ea_ref.shape[0]
    slot = (dst_ref[0, 0, :] % 8).reshape(be, 1)
    col = lax.broadcasted_iota(jnp.int32, (be, HID), 1)
    exhc = _mm(ex16, pk_ref[...])
    ex_ref[...] = jnp.where((col // HEAD_DIM) == slot, exhc, 0.0)


def _edge_math(edge_attr, QD, KN, VN, dst, Wek, bek, Wev, bev, lw, lb):
    BE = 4000
    grid = N_EDGES // BE
    eye = jnp.concatenate([jnp.eye(HEADS, dtype=_F32),
                           jnp.zeros((HEADS, HEADS), _F32)], axis=1)
    sel = jnp.repeat(eye, HEAD_DIM, axis=0)
    bc = jnp.concatenate([jnp.repeat(jnp.eye(HEADS, dtype=_F32), HEAD_DIM, axis=1),
                          jnp.zeros((HEADS, HID), _F32)], axis=0)
    colv = jnp.arange(HID) % (2 * HEADS)
    pk = (colv[None, :] == jnp.arange(2 * HEADS)[:, None]).astype(_F32)
    pk = pk.at[HEADS:].set(0.0)
    row_spec = pl.BlockSpec((BE, HID), lambda i: (i, 0))
    w_spec = pl.BlockSpec((HID, HID), lambda i: (0, 0))
    b_spec = pl.BlockSpec((1, HID), lambda i: (0, 0))
    return pl.pallas_call(
        _edge_body,
        grid=(grid,),
        in_specs=[row_spec, row_spec, row_spec, row_spec,
                  pl.BlockSpec((1, 1, BE), lambda i: (i, 0, 0)),
                  w_spec, b_spec, w_spec, b_spec, b_spec, b_spec,
                  pl.BlockSpec((HID, 2 * HEADS), lambda i: (0, 0)),
                  pl.BlockSpec((2 * HEADS, HID), lambda i: (0, 0)),
                  pl.BlockSpec((2 * HEADS, HID), lambda i: (0, 0))],
        out_specs=(row_spec, row_spec),
        out_shape=(jax.ShapeDtypeStruct((N_EDGES, HID), _F32),
                   jax.ShapeDtypeStruct((N_EDGES, HID), _F32)),
    )(edge_attr, QD, KN, VN, dst.reshape(grid, 1, BE), Wek, bek.reshape(1, HID),
      Wev, bev.reshape(1, HID), lw.reshape(1, HID), lb.reshape(1, HID),
      sel, bc, pk)


DPAD = NPAD // 8
DROWS_PER_TILE = DPAD // NS


def _sc_scatter(MSG, EX, dst):
    mesh = plsc.VectorSubcoreMesh(core_axis_name="c", subcore_axis_name="s")

    @functools.partial(
        pl.kernel,
        out_type=(jax.ShapeDtypeStruct((NC, NPAD, HID), _F32),
                  jax.ShapeDtypeStruct((NC, DPAD, HID), _F32)),
        mesh=mesh,
        scratch_types=[
            pltpu.VMEM((EPW,), jnp.int32),
            pltpu.VMEM((SCH,), jnp.int32),
            pltpu.VMEM((SCH,), jnp.int32),
            pltpu.VMEM((SCH,), jnp.int32),
            pltpu.VMEM((SCH, HID), _F32),
            pltpu.VMEM((SCH, HID), _F32),
            pltpu.VMEM_SHARED((NPAD, HID), _F32),
            pltpu.VMEM_SHARED((DPAD, HID), _F32),
            pltpu.SemaphoreType.DMA,
            pltpu.SemaphoreType.DMA,
        ],
    )
    def scatter_kernel(msg_hbm, ex_hbm, dst_hbm, on_hbm, od_hbm,
                       da_v, di_v, dp_v, zi_v, m_v, e_v, accn_s, accd_s,
                       sem, sem2):
        cid = lax.axis_index("c")
        sid = lax.axis_index("s")
        wid = sid * NC + cid

        zeros16 = jnp.zeros((16,), _F32)

        def zrow(i, carry):
            for j in range(HID // 16):
                m_v[i, pl.ds(j * 16, 16)] = zeros16
                e_v[i, pl.ds(j * 16, 16)] = zeros16
            return carry

        off = sid * ROWS_PER_TILE
        doff = sid * DROWS_PER_TILE
        lax.fori_loop(0, SCH, zrow, 0)

        iota16 = lax.iota(jnp.int32, 16)

        def _fill_zidx(base):
            for t in range(SCH // 16):
                zi_v[pl.ds(t * 16, 16)] = iota16 + (base + t * 16)

        for j in range(ROWS_PER_TILE // SCH):
            _fill_zidx(off + j * SCH)
            pltpu.sync_copy(m_v, accn_s.at[zi_v])
        _fill_zidx(doff)
        pltpu.sync_copy(e_v, accd_s.at[zi_v])
        base0 = wid * EPW
        pltpu.sync_copy(dst_hbm.at[pl.ds(base0, EPW)], da_v)
        plsc.subcore_barrier()

        def body(i, carry):
            base = base0 + i * SCH
            cm = pltpu.async_copy(msg_hbm.at[pl.ds(base, SCH)], m_v, sem)
            ce = pltpu.async_copy(ex_hbm.at[pl.ds(base, SCH)], e_v, sem2)
            for t in range(SCH // 16):
                d16 = da_v[pl.ds(i * SCH + t * 16, 16)]
                di_v[pl.ds(t * 16, 16)] = d16
                dp_v[pl.ds(t * 16, 16)] = lax.shift_right_logical(d16, 3)
            cm.wait()
            am = pltpu.async_copy(m_v, accn_s.at[di_v], sem, add=True)
            ce.wait()
            ae = pltpu.async_copy(e_v, accd_s.at[dp_v], sem2, add=True)
            am.wait()
            ae.wait()
            return carry

        lax.fori_loop(0, NSCHUNK, body, 0)
        plsc.subcore_barrier()

        for j in range(ROWS_PER_TILE // SCH):
            _fill_zidx(off + j * SCH)
            pltpu.async_copy(accn_s.at[zi_v], m_v, sem).wait()
            pltpu.sync_copy(m_v, on_hbm.at[cid, pl.ds(off + j * SCH, SCH)])
        _fill_zidx(doff)
        pltpu.async_copy(accd_s.at[zi_v], e_v, sem).wait()
        pltpu.sync_copy(e_v, od_hbm.at[cid, pl.ds(doff, SCH)])

    return scatter_kernel(MSG, EX, dst)


def _final_body(xn_ref, pn_ref, pd_ref, w1_ref, b1_ref, w2_ref, b2_ref,
                lw_ref, lb_ref, bc_ref, out_ref):
    num = pn_ref[0] + pn_ref[1]
    den = pd_ref[0] + pd_ref[1]
    denb = _mm(den, bc_ref[...])
    x_dst = xn_ref[...] + num / (denb + 1e-16)
    h = _ln(x_dst, lw_ref[...], lb_ref[...])
    t = jnp.maximum(_mm_t(h, w1_ref[...]) + b1_ref[...], 0.0)
    out_ref[...] = x_dst + _mm_t(t, w2_ref[...]) + b2_ref[...]


def _finalize(x_n, PN, PD, W1, b1, W2, b2, lw, lb):
    BN = 2000
    grid = N_NODES // BN
    DW = 2 * HEADS
    bc = jnp.concatenate([jnp.repeat(jnp.eye(HEADS, dtype=_F32), HEAD_DIM, axis=1),
                          jnp.zeros((HEADS, HID), _F32)], axis=0)
    return pl.pallas_call(
        _final_body,
        grid=(grid,),
        in_specs=[pl.BlockSpec((BN, HID), lambda i: (i, 0)),
                  pl.BlockSpec((NC, BN, HID), lambda i: (0, i, 0)),
                  pl.BlockSpec((NC, BN, DW), lambda i: (0, i, 0)),
                  pl.BlockSpec((4 * HID, HID), lambda i: (0, 0)),
                  pl.BlockSpec((1, 4 * HID), lambda i: (0, 0)),
                  pl.BlockSpec((HID, 4 * HID), lambda i: (0, 0)),
                  pl.BlockSpec((1, HID), lambda i: (0, 0)),
                  pl.BlockSpec((1, HID), lambda i: (0, 0)),
                  pl.BlockSpec((1, HID), lambda i: (0, 0)),
                  pl.BlockSpec((DW, HID), lambda i: (0, 0))],
        out_specs=pl.BlockSpec((BN, HID), lambda i: (i, 0)),
        out_shape=jax.ShapeDtypeStruct((N_NODES, HID), _F32),
    )(x_n, PN, PD, W1, b1.reshape(1, 4 * HID), W2, b2.reshape(1, HID),
      lw.reshape(1, HID), lb.reshape(1, HID), bc)


def kernel(x, edge_index, edge_attr, Wq, bq, Wk, bk, Wv, bv, Wek, bek, Wev, bev,
           W1, b1, W2, b2, ln_src_w, ln_src_b, ln_edge_w, ln_edge_b,
           ln_ffn_w, ln_ffn_b):
    src = edge_index[0].astype(jnp.int32)
    dst = edge_index[1].astype(jnp.int32)
    x_n, Qs, K, V = _node_prep(x, Wq, bq, Wk, bk, Wv, bv, ln_src_w, ln_src_b)
    QD, KN, VN = _sc_gather(Qs, K, V, src, dst)
    MSG, EX = _edge_math(edge_attr, QD, KN, VN, dst, Wek, bek, Wev, bev,
                         ln_edge_w, ln_edge_b)
    PN, PD2 = _sc_scatter(MSG, EX, dst)
    PD = PD2.reshape(NC, NPAD, 2 * HEADS)
    return _finalize(x_n, PN, PD, W1, b1, W2, b2, ln_ffn_w, ln_ffn_b)


# speedup vs baseline: 39.7203x; 1.0677x over previous
import functools

import jax
import jax.numpy as jnp
from jax import lax
from jax.experimental import pallas as pl
from jax.experimental.pallas import tpu as pltpu
from jax.experimental.pallas import tpu_sc as plsc

HID = 128
HEADS = 8
HEAD_DIM = HID // HEADS
N_NODES = 10000
N_EDGES = 320000

NC = 2
NS = 16
NW = NC * NS
EPW = N_EDGES // NW
CH = 80
NCHUNK = EPW // CH
SCH = 80
NSCHUNK = EPW // SCH
NPAD = 10240
ROWS_PER_TILE = NPAD // NS

_F32 = jnp.float32
_HIGH = jax.lax.Precision.DEFAULT


def _mm_t(a, w):
    return jax.lax.dot_general(
        a, w, (((1,), (1,)), ((), ())),
        precision=_HIGH, preferred_element_type=_F32)


def _mm(a, b):
    return jax.lax.dot_general(
        a, b, (((1,), (0,)), ((), ())),
        precision=_HIGH, preferred_element_type=_F32)


def _ln(x, w, b):
    mu = jnp.mean(x, axis=-1, keepdims=True)
    var = jnp.mean((x - mu) ** 2, axis=-1, keepdims=True)
    return (x - mu) / jnp.sqrt(var + 1e-5) * w + b


def _node_prep_body(x_ref, wq_ref, bq_ref, wk_ref, bk_ref, wv_ref, bv_ref,
                    lw_ref, lb_ref, xn_ref, q_ref, k_ref, v_ref):
    xn = _ln(x_ref[...], lw_ref[...], lb_ref[...])
    xn_ref[...] = xn
    q_ref[...] = (_mm_t(xn, wq_ref[...]) + bq_ref[...]) * (1.0 / jnp.sqrt(jnp.float32(HEAD_DIM)))
    k_ref[...] = _mm_t(xn, wk_ref[...]) + bk_ref[...]
    v_ref[...] = _mm_t(xn, wv_ref[...]) + bv_ref[...]


def _node_prep(x, Wq, bq, Wk, bk, Wv, bv, lw, lb):
    out = jax.ShapeDtypeStruct((N_NODES, HID), _F32)
    return pl.pallas_call(
        _node_prep_body,
        out_shape=(out, out, out, out),
    )(x, Wq, bq.reshape(1, HID), Wk, bk.reshape(1, HID),
      Wv, bv.reshape(1, HID), lw.reshape(1, HID), lb.reshape(1, HID))


def _sc_gather(Qs, K, V, src, dst):
    mesh = plsc.VectorSubcoreMesh(core_axis_name="c", subcore_axis_name="s")
    erows = jax.ShapeDtypeStruct((N_EDGES, HID), _F32)

    @functools.partial(
        pl.kernel,
        out_type=(erows, erows, erows),
        mesh=mesh,
        scratch_types=[
            pltpu.VMEM((EPW,), jnp.int32),
            pltpu.VMEM((EPW,), jnp.int32),
            [pltpu.VMEM((CH,), jnp.int32)] * 4,
            [pltpu.VMEM((CH, HID), _F32)] * 6,
            [pltpu.SemaphoreType.DMA] * 6,
        ],
    )
    def gather_kernel(q_hbm, k_hbm, v_hbm, src_hbm, dst_hbm,
                      qd_hbm, kn_hbm, vn_hbm, si_v, di_v, idx, bufs, sems):
        wid = lax.axis_index("s") * NC + lax.axis_index("c")
        base0 = wid * EPW
        pltpu.sync_copy(src_hbm.at[pl.ds(base0, EPW)], si_v)
        pltpu.sync_copy(dst_hbm.at[pl.ds(base0, EPW)], di_v)

        def one_chunk(i, sc_v, dc_v, bq_v, bk_v, bv_v, sq, sk, sv):
            for t in range(CH // 16):
                sc_v[pl.ds(t * 16, 16)] = si_v[pl.ds(i * CH + t * 16, 16)]
                dc_v[pl.ds(t * 16, 16)] = di_v[pl.ds(i * CH + t * 16, 16)]
            return (pltpu.async_copy(q_hbm.at[dc_v], bq_v, sq),
                    pltpu.async_copy(k_hbm.at[sc_v], bk_v, sk),
                    pltpu.async_copy(v_hbm.at[sc_v], bv_v, sv))

        def body(p, carry):
            a = 2 * p
            b = 2 * p + 1
            csa = one_chunk(a, idx[0], idx[1], bufs[0], bufs[1], bufs[2],
                            sems[0], sems[1], sems[2])
            csb = one_chunk(b, idx[2], idx[3], bufs[3], bufs[4], bufs[5],
                            sems[3], sems[4], sems[5])
            basea = base0 + a * CH
            baseb = base0 + b * CH
            csa[0].wait()
            wa0 = pltpu.async_copy(bufs[0], qd_hbm.at[pl.ds(basea, CH)], sems[0])
            csa[1].wait()
            wa1 = pltpu.async_copy(bufs[1], kn_hbm.at[pl.ds(basea, CH)], sems[1])
            csa[2].wait()
            wa2 = pltpu.async_copy(bufs[2], vn_hbm.at[pl.ds(basea, CH)], sems[2])
            csb[0].wait()
            wb0 = pltpu.async_copy(bufs[3], qd_hbm.at[pl.ds(baseb, CH)], sems[3])
            csb[1].wait()
            wb1 = pltpu.async_copy(bufs[4], kn_hbm.at[pl.ds(baseb, CH)], sems[4])
            csb[2].wait()
            wb2 = pltpu.async_copy(bufs[5], vn_hbm.at[pl.ds(baseb, CH)], sems[5])
            for w in (wa0, wa1, wa2, wb0, wb1, wb2):
                w.wait()
            return carry

        lax.fori_loop(0, NCHUNK // 2, body, 0)

        i = NCHUNK - 1
        cs = one_chunk(i, idx[0], idx[1], bufs[0], bufs[1], bufs[2],
                       sems[0], sems[1], sems[2])
        base = base0 + i * CH
        cs[0].wait()
        pltpu.sync_copy(bufs[0], qd_hbm.at[pl.ds(base, CH)])
        cs[1].wait()
        pltpu.sync_copy(bufs[1], kn_hbm.at[pl.ds(base, CH)])
        cs[2].wait()
        pltpu.sync_copy(bufs[2], vn_hbm.at[pl.ds(base, CH)])

    return gather_kernel(Qs, K, V, src, dst)


def _edge_body(ea_ref, qd_ref, kn_ref, vn_ref, dst_ref, wek_ref, bek_ref,
               wev_ref, bev_ref, lw_ref, lb_ref, sel_ref, bc_ref, pk_ref,
               msg_ref, ex_ref):
    en = _ln(ea_ref[...], lw_ref[...], lb_ref[...])
    ek = _mm_t(en, wek_ref[...]) + bek_ref[...]
    prod = qd_ref[...] * (kn_ref[...] + ek)
    ex16 = jnp.exp(_mm(prod, sel_ref[...]))
    ev = _mm_t(en, wev_ref[...]) + bev_ref[...]
    exb = _mm(ex16, bc_ref[...])
    msg_ref[...] = (vn_ref[...] + ev) * exb
    be = ea_ref.shape[0]
    slot = (dst_ref[0, 0, :] % 8).reshape(be, 1)
    col = lax.broadcasted_iota(jnp.int32, (be, HID), 1)
    exhc = _mm(ex16, pk_ref[...])
    ex_ref[...] = jnp.where((col // HEAD_DIM) == slot, exhc, 0.0)


def _edge_math(edge_attr, QD, KN, VN, dst, Wek, bek, Wev, bev, lw, lb):
    BE = 4000
    grid = N_EDGES // BE
    eye = jnp.concatenate([jnp.eye(HEADS, dtype=_F32),
                           jnp.zeros((HEADS, HEADS), _F32)], axis=1)
    sel = jnp.repeat(eye, HEAD_DIM, axis=0)
    bc = jnp.concatenate([jnp.repeat(jnp.eye(HEADS, dtype=_F32), HEAD_DIM, axis=1),
                          jnp.zeros((HEADS, HID), _F32)], axis=0)
    colv = jnp.arange(HID) % (2 * HEADS)
    pk = (colv[None, :] == jnp.arange(2 * HEADS)[:, None]).astype(_F32)
    pk = pk.at[HEADS:].set(0.0)
    row_spec = pl.BlockSpec((BE, HID), lambda i: (i, 0))
    w_spec = pl.BlockSpec((HID, HID), lambda i: (0, 0))
    b_spec = pl.BlockSpec((1, HID), lambda i: (0, 0))
    return pl.pallas_call(
        _edge_body,
        grid=(grid,),
        in_specs=[row_spec, row_spec, row_spec, row_spec,
                  pl.BlockSpec((1, 1, BE), lambda i: (i, 0, 0)),
                  w_spec, b_spec, w_spec, b_spec, b_spec, b_spec,
                  pl.BlockSpec((HID, 2 * HEADS), lambda i: (0, 0)),
                  pl.BlockSpec((2 * HEADS, HID), lambda i: (0, 0)),
                  pl.BlockSpec((2 * HEADS, HID), lambda i: (0, 0))],
        out_specs=(row_spec, row_spec),
        out_shape=(jax.ShapeDtypeStruct((N_EDGES, HID), _F32),
                   jax.ShapeDtypeStruct((N_EDGES, HID), _F32)),
    )(edge_attr, QD, KN, VN, dst.reshape(grid, 1, BE), Wek, bek.reshape(1, HID),
      Wev, bev.reshape(1, HID), lw.reshape(1, HID), lb.reshape(1, HID),
      sel, bc, pk)


DPAD = NPAD // 8
DROWS_PER_TILE = DPAD // NS


def _sc_scatter(MSG, EX, dst):
    mesh = plsc.VectorSubcoreMesh(core_axis_name="c", subcore_axis_name="s")

    @functools.partial(
        pl.kernel,
        out_type=(jax.ShapeDtypeStruct((NC, NPAD, HID), _F32),
                  jax.ShapeDtypeStruct((NC, DPAD, HID), _F32)),
        mesh=mesh,
        scratch_types=[
            pltpu.VMEM((EPW,), jnp.int32),
            pltpu.VMEM((SCH,), jnp.int32),
            pltpu.VMEM((SCH,), jnp.int32),
            pltpu.VMEM((SCH,), jnp.int32),
            pltpu.VMEM((SCH, HID), _F32),
            pltpu.VMEM((SCH, HID), _F32),
            pltpu.VMEM_SHARED((NPAD, HID), _F32),
            pltpu.VMEM_SHARED((DPAD, HID), _F32),
            pltpu.SemaphoreType.DMA,
            pltpu.SemaphoreType.DMA,
        ],
    )
    def scatter_kernel(msg_hbm, ex_hbm, dst_hbm, on_hbm, od_hbm,
                       da_v, di_v, dp_v, zi_v, m_v, e_v, accn_s, accd_s,
                       sem, sem2):
        cid = lax.axis_index("c")
        sid = lax.axis_index("s")
        wid = sid * NC + cid

        zeros16 = jnp.zeros((16,), _F32)

        def zrow(i, carry):
            for j in range(HID // 16):
                m_v[i, pl.ds(j * 16, 16)] = zeros16
                e_v[i, pl.ds(j * 16, 16)] = zeros16
            return carry

        off = sid * ROWS_PER_TILE
        doff = sid * DROWS_PER_TILE
        lax.fori_loop(0, SCH, zrow, 0)

        iota16 = lax.iota(jnp.int32, 16)

        def _fill_zidx(base):
            for t in range(SCH // 16):
                zi_v[pl.ds(t * 16, 16)] = iota16 + (base + t * 16)

        for j in range(ROWS_PER_TILE // SCH):
            _fill_zidx(off + j * SCH)
            pltpu.sync_copy(m_v, accn_s.at[zi_v])
        _fill_zidx(doff)
        pltpu.sync_copy(e_v, accd_s.at[zi_v])
        base0 = wid * EPW
        pltpu.sync_copy(dst_hbm.at[pl.ds(base0, EPW)], da_v)
        plsc.subcore_barrier()

        def body(i, carry):
            base = base0 + i * SCH
            cm = pltpu.async_copy(msg_hbm.at[pl.ds(base, SCH)], m_v, sem)
            ce = pltpu.async_copy(ex_hbm.at[pl.ds(base, SCH)], e_v, sem2)
            for t in range(SCH // 16):
                d16 = da_v[pl.ds(i * SCH + t * 16, 16)]
                di_v[pl.ds(t * 16, 16)] = d16
                dp_v[pl.ds(t * 16, 16)] = lax.shift_right_logical(d16, 3)
            cm.wait()
            am = pltpu.async_copy(m_v, accn_s.at[di_v], sem, add=True)
            ce.wait()
            ae = pltpu.async_copy(e_v, accd_s.at[dp_v], sem2, add=True)
            am.wait()
            ae.wait()
            return carry

        lax.fori_loop(0, NSCHUNK, body, 0)
        plsc.subcore_barrier()

        for j in range(ROWS_PER_TILE // SCH):
            _fill_zidx(off + j * SCH)
            pltpu.async_copy(accn_s.at[zi_v], m_v, sem).wait()
            pltpu.sync_copy(m_v, on_hbm.at[cid, pl.ds(off + j * SCH, SCH)])
        _fill_zidx(doff)
        pltpu.async_copy(accd_s.at[zi_v], e_v, sem).wait()
        pltpu.sync_copy(e_v, od_hbm.at[cid, pl.ds(doff, SCH)])

    return scatter_kernel(MSG, EX, dst)


def _final_body(xn_ref, pn_ref, pd_ref, w1_ref, b1_ref, w2_ref, b2_ref,
                lw_ref, lb_ref, bc_ref, out_ref):
    num = pn_ref[0] + pn_ref[1]
    den = pd_ref[0] + pd_ref[1]
    denb = _mm(den, bc_ref[...])
    x_dst = xn_ref[...] + num / (denb + 1e-16)
    h = _ln(x_dst, lw_ref[...], lb_ref[...])
    t = jnp.maximum(_mm_t(h, w1_ref[...]) + b1_ref[...], 0.0)
    out_ref[...] = x_dst + _mm_t(t, w2_ref[...]) + b2_ref[...]


def _finalize(x_n, PN, PD, W1, b1, W2, b2, lw, lb):
    BN = 2000
    grid = N_NODES // BN
    DW = 2 * HEADS
    bc = jnp.concatenate([jnp.repeat(jnp.eye(HEADS, dtype=_F32), HEAD_DIM, axis=1),
                          jnp.zeros((HEADS, HID), _F32)], axis=0)
    return pl.pallas_call(
        _final_body,
        grid=(grid,),
        in_specs=[pl.BlockSpec((BN, HID), lambda i: (i, 0)),
                  pl.BlockSpec((NC, BN, HID), lambda i: (0, i, 0)),
                  pl.BlockSpec((NC, BN, DW), lambda i: (0, i, 0)),
                  pl.BlockSpec((4 * HID, HID), lambda i: (0, 0)),
                  pl.BlockSpec((1, 4 * HID), lambda i: (0, 0)),
                  pl.BlockSpec((HID, 4 * HID), lambda i: (0, 0)),
                  pl.BlockSpec((1, HID), lambda i: (0, 0)),
                  pl.BlockSpec((1, HID), lambda i: (0, 0)),
                  pl.BlockSpec((1, HID), lambda i: (0, 0)),
                  pl.BlockSpec((DW, HID), lambda i: (0, 0))],
        out_specs=pl.BlockSpec((BN, HID), lambda i: (i, 0)),
        out_shape=jax.ShapeDtypeStruct((N_NODES, HID), _F32),
    )(x_n, PN, PD, W1, b1.reshape(1, 4 * HID), W2, b2.reshape(1, HID),
      lw.reshape(1, HID), lb.reshape(1, HID), bc)


def kernel(x, edge_index, edge_attr, Wq, bq, Wk, bk, Wv, bv, Wek, bek, Wev, bev,
           W1, b1, W2, b2, ln_src_w, ln_src_b, ln_edge_w, ln_edge_b,
           ln_ffn_w, ln_ffn_b):
    src = edge_index[0].astype(jnp.int32)
    dst = edge_index[1].astype(jnp.int32)
    x_n, Qs, K, V = _node_prep(x, Wq, bq, Wk, bk, Wv, bv, ln_src_w, ln_src_b)
    QD, KN, VN = _sc_gather(Qs, K, V, src, dst)
    MSG, EX = _edge_math(edge_attr, QD, KN, VN, dst, Wek, bek, Wev, bev,
                         ln_edge_w, ln_edge_b)
    PN, PD2 = _sc_scatter(MSG, EX, dst)
    PD = PD2.reshape(NC, NPAD, 2 * HEADS)
    return _finalize(x_n, PN, PD, W1, b1, W2, b2, ln_ffn_w, ln_ffn_b)
